# baseline (speedup 1.0000x reference)
import functools

import jax
import jax.numpy as jnp
from jax import lax
from jax.experimental import pallas as pl
from jax.experimental.pallas import tpu as pltpu
from jax.experimental.pallas import tpu_sc as plsc

TOTAL = 32768
D = 512
NSEG = 16
NCORES = 2
NSUB = 16
NW = NCORES * NSUB
TCR = 22528
TC_BLOCK = 1024
TC_GRID = TCR // TC_BLOCK
SC_BASE = TCR
RPW = (TOTAL - SC_BASE) // NW
CHUNK = 96
NVEC = D // 16
MAXCH = RPW // CHUNK + NSEG + 1

_mesh = plsc.VectorSubcoreMesh(core_axis_name="c", subcore_axis_name="s")


@functools.partial(
    pl.kernel,
    out_type=jax.ShapeDtypeStruct((NW, NSEG, D), jnp.float32),
    mesh=_mesh,
    scratch_types=[
        pltpu.VMEM((24,), jnp.int32),
        pltpu.VMEM((CHUNK, D), jnp.float32),
        pltpu.VMEM((CHUNK, D), jnp.float32),
        pltpu.VMEM((NSEG, D), jnp.float32),
        pltpu.SMEM((MAXCH,), jnp.int32),
        pltpu.SMEM((MAXCH,), jnp.int32),
        pltpu.SMEM((MAXCH,), jnp.int32),
        pltpu.SMEM((MAXCH,), jnp.int32),
        pltpu.SemaphoreType.DMA,
        pltpu.SemaphoreType.DMA,
    ],
)
def _seg_partials(slices_hbm, in_hbm, out_hbm, sl_v, buf_a, buf_b, acc_ref,
                  d_base, d_low, d_cap, d_seg, sem_a, sem_b):
    wid = lax.axis_index("c") * NSUB + lax.axis_index("s")
    lo = SC_BASE + wid * RPW
    hi = lo + RPW
    pltpu.sync_copy(slices_hbm, sl_v.at[pl.ds(0, NSEG + 1)])
    va = sl_v[pl.ds(0, 16)]
    vb = sl_v[pl.ds(8, 16)]
    svals = [va[i] for i in range(16)] + [vb[8]]

    nch = jnp.int32(0)
    for i in range(NSEG):
        a = jnp.minimum(jnp.maximum(svals[i], lo), hi)
        b = jnp.minimum(jnp.maximum(svals[i + 1], lo), hi)
        a0 = lax.div(a, 8) * 8
        nc_i = jnp.where(b > a, lax.div(b - a0 + CHUNK - 1, CHUNK), 0)

        def desc_body(k, n, a=a, b=b, a0=a0, i=i):
            base0 = a0 + k * CHUNK
            d_base[n] = lax.div(jnp.minimum(base0, TOTAL - CHUNK), 8)
            d_low[n] = jnp.maximum(a, base0)
            d_cap[n] = jnp.minimum(b, base0 + CHUNK)
            d_seg[n] = jnp.int32(i)
            return n + 1

        nch = lax.fori_loop(0, nc_i, desc_body, nch)

    zvec = jnp.zeros((16,), jnp.float32)

    def zero_body(i, _):
        for j in range(NVEC):
            acc_ref[i, pl.ds(16 * j, 16)] = zvec
        return 0

    lax.fori_loop(0, NSEG, zero_body, 0)

    def start(k, buf, sem):
        pltpu.async_copy(in_hbm.at[pl.ds(d_base[k] * 8, CHUNK)], buf, sem)

    def wait(buf, sem):
        pltpu.make_async_copy(in_hbm.at[pl.ds(0, CHUNK)], buf, sem).wait()

    def process(k, buf):
        low = d_low[k]
        cap = d_cap[k]
        seg = d_seg[k]
        base = d_base[k] * 8

        def row_body(r, accs):
            g = base + r
            w = jnp.where((g >= low) & (g < cap), 1.0, 0.0)
            return tuple(
                accs[j] + w * buf[r, pl.ds(16 * j, 16)] for j in range(NVEC)
            )

        accs = tuple(zvec for _ in range(NVEC))
        accs = lax.fori_loop(0, CHUNK, row_body, accs)
        for j in range(NVEC):
            plsc.addupdate(acc_ref.at[seg, pl.ds(16 * j, 16)], accs[j])

    @pl.when(nch > 0)
    def _():
        start(0, buf_a, sem_a)

    def pair_body(m, _):
        k0 = 2 * m
        k1 = k0 + 1

        @pl.when(k1 < nch)
        def _():
            start(k1, buf_b, sem_b)

        wait(buf_a, sem_a)
        process(k0, buf_a)

        @pl.when(k1 + 1 < nch)
        def _():
            start(k1 + 1, buf_a, sem_a)

        @pl.when(k1 < nch)
        def _():
            wait(buf_b, sem_b)
            process(k1, buf_b)

        return 0

    lax.fori_loop(0, lax.div(nch + 1, 2), pair_body, 0)

    pltpu.sync_copy(acc_ref, out_hbm.at[wid])


def _tc_body(s_ref, x_ref, o_ref):
    pid = pl.program_id(0)
    gio = lax.broadcasted_iota(jnp.int32, (NSEG, TC_BLOCK), 1) + pid * TC_BLOCK
    sio = lax.broadcasted_iota(jnp.int32, (NSEG, TC_BLOCK), 0)
    starts = jnp.zeros((NSEG, TC_BLOCK), jnp.int32)
    ends = starts
    for i in range(NSEG):
        starts = jnp.where(sio == i, s_ref[i], starts)
        ends = jnp.where(sio == i, s_ref[i + 1], ends)
    m = ((gio >= starts) & (gio < ends)).astype(jnp.float32)
    acc = jnp.dot(m, x_ref[...], preferred_element_type=jnp.float32)

    @pl.when(pid == 0)
    def _():
        o_ref[...] = acc

    @pl.when(pid != 0)
    def _():
        o_ref[...] += acc


def _tc_segsum(slices, x):
    return pl.pallas_call(
        _tc_body,
        grid_spec=pltpu.PrefetchScalarGridSpec(
            num_scalar_prefetch=1,
            grid=(TC_GRID,),
            in_specs=[pl.BlockSpec((TC_BLOCK, D), lambda i, *_: (i, 0))],
            out_specs=pl.BlockSpec((NSEG, D), lambda i, *_: (0, 0)),
        ),
        out_shape=jax.ShapeDtypeStruct((NSEG, D), jnp.float32),
    )(slices, x)


def _combine_body(p_ref, t_ref, o_ref):
    o_ref[...] = jnp.sum(p_ref[...], axis=0) + t_ref[...]


def _combine(partials, tc_out):
    return pl.pallas_call(
        _combine_body,
        out_shape=jax.ShapeDtypeStruct((NSEG, D), jnp.float32),
    )(partials, tc_out)


def kernel(slices, inputs):
    partials = _seg_partials(slices, inputs)
    tc_out = _tc_segsum(slices, inputs)
    return _combine(partials, tc_out)


# speedup vs baseline: 1.0150x; 1.0150x over previous
import functools

import jax
import jax.numpy as jnp
from jax import lax
from jax.experimental import pallas as pl
from jax.experimental.pallas import tpu as pltpu
from jax.experimental.pallas import tpu_sc as plsc

TOTAL = 32768
D = 512
NSEG = 16
NCORES = 2
NSUB = 16
NW = NCORES * NSUB
TCR = 22528
TC_BLOCK = 1024
TC_GRID = TCR // TC_BLOCK
SC_BASE = TCR
RPW = (TOTAL - SC_BASE) // NW
CHUNK = 64
NVEC = D // 16
MAXCH = RPW // CHUNK + NSEG + 1

_mesh = plsc.VectorSubcoreMesh(core_axis_name="c", subcore_axis_name="s")


@functools.partial(
    pl.kernel,
    out_type=jax.ShapeDtypeStruct((NW, NSEG, D), jnp.float32),
    mesh=_mesh,
    scratch_types=[
        pltpu.VMEM((24,), jnp.int32),
        pltpu.VMEM((CHUNK, D), jnp.float32),
        pltpu.VMEM((CHUNK, D), jnp.float32),
        pltpu.VMEM((NSEG, D), jnp.float32),
        pltpu.SMEM((MAXCH,), jnp.int32),
        pltpu.SMEM((MAXCH,), jnp.int32),
        pltpu.SMEM((MAXCH,), jnp.int32),
        pltpu.SMEM((MAXCH,), jnp.int32),
        pltpu.SemaphoreType.DMA,
        pltpu.SemaphoreType.DMA,
    ],
)
def _seg_partials(slices_hbm, in_hbm, out_hbm, sl_v, buf_a, buf_b, acc_ref,
                  d_base, d_low, d_cap, d_seg, sem_a, sem_b):
    wid = lax.axis_index("c") * NSUB + lax.axis_index("s")
    lo = SC_BASE + wid * RPW
    hi = lo + RPW
    pltpu.sync_copy(slices_hbm, sl_v.at[pl.ds(0, NSEG + 1)])
    va = sl_v[pl.ds(0, 16)]
    vb = sl_v[pl.ds(8, 16)]
    svals = [va[i] for i in range(16)] + [vb[8]]

    nch = jnp.int32(0)
    for i in range(NSEG):
        a = jnp.minimum(jnp.maximum(svals[i], lo), hi)
        b = jnp.minimum(jnp.maximum(svals[i + 1], lo), hi)
        a0 = lax.div(a, 8) * 8
        nc_i = jnp.where(b > a, lax.div(b - a0 + CHUNK - 1, CHUNK), 0)

        def desc_body(k, n, a=a, b=b, a0=a0, i=i):
            base0 = a0 + k * CHUNK
            d_base[n] = lax.div(jnp.minimum(base0, TOTAL - CHUNK), 8)
            d_low[n] = jnp.maximum(a, base0)
            d_cap[n] = jnp.minimum(b, base0 + CHUNK)
            d_seg[n] = jnp.int32(i)
            return n + 1

        nch = lax.fori_loop(0, nc_i, desc_body, nch)

    zvec = jnp.zeros((16,), jnp.float32)

    def zero_body(i, _):
        for j in range(NVEC):
            acc_ref[i, pl.ds(16 * j, 16)] = zvec
        return 0

    lax.fori_loop(0, NSEG, zero_body, 0)

    def start(k, buf, sem):
        pltpu.async_copy(in_hbm.at[pl.ds(d_base[k] * 8, CHUNK)], buf, sem)

    def wait(buf, sem):
        pltpu.make_async_copy(in_hbm.at[pl.ds(0, CHUNK)], buf, sem).wait()

    def process(k, buf):
        low = d_low[k]
        cap = d_cap[k]
        seg = d_seg[k]
        base = d_base[k] * 8

        def row_body(r, accs):
            g = base + r
            w = jnp.where((g >= low) & (g < cap), 1.0, 0.0)
            return tuple(
                accs[j] + w * buf[r, pl.ds(16 * j, 16)] for j in range(NVEC)
            )

        accs = tuple(zvec for _ in range(NVEC))
        accs = lax.fori_loop(0, CHUNK, row_body, accs)
        for j in range(NVEC):
            plsc.addupdate(acc_ref.at[seg, pl.ds(16 * j, 16)], accs[j])

    @pl.when(nch > 0)
    def _():
        start(0, buf_a, sem_a)

    def pair_body(m, _):
        k0 = 2 * m
        k1 = k0 + 1

        @pl.when(k1 < nch)
        def _():
            start(k1, buf_b, sem_b)

        wait(buf_a, sem_a)
        process(k0, buf_a)

        @pl.when(k1 + 1 < nch)
        def _():
            start(k1 + 1, buf_a, sem_a)

        @pl.when(k1 < nch)
        def _():
            wait(buf_b, sem_b)
            process(k1, buf_b)

        return 0

    lax.fori_loop(0, lax.div(nch + 1, 2), pair_body, 0)

    pltpu.sync_copy(acc_ref, out_hbm.at[wid])


def _tc_body(s_ref, x_ref, o_ref):
    pid = pl.program_id(0)
    gio = lax.broadcasted_iota(jnp.int32, (NSEG, TC_BLOCK), 1) + pid * TC_BLOCK
    sio = lax.broadcasted_iota(jnp.int32, (NSEG, TC_BLOCK), 0)
    starts = jnp.zeros((NSEG, TC_BLOCK), jnp.int32)
    ends = starts
    for i in range(NSEG):
        starts = jnp.where(sio == i, s_ref[i], starts)
        ends = jnp.where(sio == i, s_ref[i + 1], ends)
    m = ((gio >= starts) & (gio < ends)).astype(jnp.float32)
    acc = jnp.dot(m, x_ref[...], preferred_element_type=jnp.float32)

    @pl.when(pid == 0)
    def _():
        o_ref[...] = acc

    @pl.when(pid != 0)
    def _():
        o_ref[...] += acc


def _tc_segsum(slices, x):
    return pl.pallas_call(
        _tc_body,
        grid_spec=pltpu.PrefetchScalarGridSpec(
            num_scalar_prefetch=1,
            grid=(TC_GRID,),
            in_specs=[pl.BlockSpec((TC_BLOCK, D), lambda i, *_: (i, 0))],
            out_specs=pl.BlockSpec((NSEG, D), lambda i, *_: (0, 0)),
        ),
        out_shape=jax.ShapeDtypeStruct((NSEG, D), jnp.float32),
    )(slices, x)


def _combine_body(p_ref, t_ref, o_ref):
    o_ref[...] = jnp.sum(p_ref[...], axis=0) + t_ref[...]


def _combine(partials, tc_out):
    return pl.pallas_call(
        _combine_body,
        out_shape=jax.ShapeDtypeStruct((NSEG, D), jnp.float32),
    )(partials, tc_out)


def kernel(slices, inputs):
    partials = _seg_partials(slices, inputs)
    tc_out = _tc_segsum(slices, inputs)
    return _combine(partials, tc_out)
